# baseline (speedup 1.0000x reference)
import functools

import jax
import jax.numpy as jnp
from jax import lax
from jax.experimental import pallas as pl
from jax.experimental.pallas import tpu as pltpu
from jax.experimental.pallas import tpu_sc as plsc

N = 10000
D = 128
E = 320000
NC = 2
NS = 16
NW = NC * NS
CHUNK = 128
CPW = 80
E_PAD = NW * CPW * CHUNK
ROWS_PER_TILE = 632
N_ACC = NS * ROWS_PER_TILE
LAST_TILE_OUT = N - (NS - 1) * ROWS_PER_TILE
NBUF = 2
IDEPTH = 4

_MESH = plsc.VectorSubcoreMesh(core_axis_name="c", subcore_axis_name="s")


N_REAL_CHUNKS = E // CHUNK


def _sc_segment_sum(m, ei, zrows):
    @functools.partial(
        pl.kernel,
        mesh=_MESH,
        out_type=[jax.ShapeDtypeStruct((N, D), jnp.float32),
                  jax.ShapeDtypeStruct((N, D), jnp.float32)],
        scratch_types=[
            pltpu.VMEM((IDEPTH, CHUNK), jnp.int32),
            pltpu.VMEM((IDEPTH, CHUNK), jnp.int32),
            pltpu.VMEM((NBUF, CHUNK, D), jnp.float32),
            pltpu.VMEM_SHARED((N_ACC, D), jnp.float32),
        ]
        + [pltpu.SemaphoreType.DMA] * (2 * IDEPTH + 2 * NBUF),
    )
    def k(m_hbm, ei_hbm, z_hbm, out0_hbm, out1_hbm, cbuf, rbuf,
          rows, acc, *sems):
        csem = sems[:IDEPTH]
        rsem = sems[IDEPTH:2 * IDEPTH]
        gsem = sems[2 * IDEPTH:2 * IDEPTH + NBUF]
        ssem = sems[2 * IDEPTH + NBUF:]
        c = lax.axis_index("c")
        s = lax.axis_index("s")
        wid = c * NS + s
        row_v = ei_hbm.at[0]
        col_v = ei_hbm.at[1]
        chunk0 = wid * CPW

        def real(q):
            return chunk0 + q < N_REAL_CHUNKS

        def colload(q, i):
            return pltpu.make_async_copy(
                col_v.at[pl.ds((chunk0 + q) * CHUNK, CHUNK)], cbuf.at[i],
                csem[i])

        def rowload(q, i):
            return pltpu.make_async_copy(
                row_v.at[pl.ds((chunk0 + q) * CHUNK, CHUNK)], rbuf.at[i],
                rsem[i])


        def gather(i, b):
            return pltpu.make_async_copy(m_hbm.at[cbuf.at[i]], rows.at[b],
                                         gsem[b])

        def scatter(i, b):
            return pltpu.make_async_copy(rows.at[b], acc.at[rbuf.at[i]],
                                         ssem[b])

        with jax.named_scope("sc_zero_and_prefetch"):
            z = pltpu.make_async_copy(
                z_hbm, acc.at[pl.ds(s * ROWS_PER_TILE, ROWS_PER_TILE)],
                ssem[1])
            z.start()

            for i in range(IDEPTH):
                colload(i, i).start()
                rowload(i, i).start()

            colload(0, 0).wait()
            gather(0, 0).start()

        def _loop_body(ci):
            for j in range(IDEPTH):
                q = ci + j
                b = j % NBUF
                bp = (j + 1) % NBUF
                i = j
                ip = (j + 1) % IDEPTH

                @pl.when((q >= 1) & real(q - 1))
                def _():
                    scatter((j - 1) % IDEPTH, bp).wait()

                    @pl.when((q + 3 < CPW) & real(q + 3))
                    def _():
                        rowload(q + 3, (j - 1) % IDEPTH).start()

                @pl.when((q + 1 < CPW) & real(q + 1))
                def _():
                    colload(q + 1, ip).wait()
                    gather(ip, bp).start()

                @pl.when(real(q))
                def _():
                    gather(i, b).wait()

                @pl.when((q + 4 < CPW) & real(q + 4))
                def _():
                    colload(q + 4, i).start()

                @pl.when(real(q))
                def _():
                    rowload(q, i).wait()

                @pl.when(q == 0)
                def _():
                    pltpu.make_async_copy(
                        z_hbm,
                        acc.at[pl.ds(s * ROWS_PER_TILE, ROWS_PER_TILE)],
                        ssem[1]).wait()
                    plsc.subcore_barrier()

                @pl.when(real(q))
                def _():
                    scatter(i, b).start(add=True)

        with jax.named_scope("sc_edge_loop"):

            @pl.loop(0, CPW, step=IDEPTH)
            def _(ci):
                _loop_body(ci)

            @pl.when(real(CPW - 1))
            def _():
                scatter((CPW - 1) % IDEPTH, (CPW - 1) % NBUF).wait()

        plsc.subcore_barrier()

        with jax.named_scope("sc_copy_out"):
            for cc, out_c in ((0, out0_hbm), (1, out1_hbm)):

                @pl.when((c == cc) & (s < NS - 1))
                def _():
                    pltpu.sync_copy(
                        acc.at[pl.ds(s * ROWS_PER_TILE, ROWS_PER_TILE)],
                        out_c.at[pl.ds(s * ROWS_PER_TILE, ROWS_PER_TILE)],
                    )

                @pl.when((c == cc) & (s == NS - 1))
                def _():
                    pltpu.sync_copy(
                        acc.at[pl.ds((NS - 1) * ROWS_PER_TILE, LAST_TILE_OUT)],
                        out_c.at[pl.ds((NS - 1) * ROWS_PER_TILE,
                                       LAST_TILE_OUT)],
                    )

    return k(m, ei, zrows)


_BLK = 2000


def _tc_msg(h, wc):
    def body(h_ref, w_ref, o_ref):
        o_ref[...] = jnp.maximum(
            lax.dot_general(h_ref[...], w_ref[...],
                            (((1,), (1,)), ((), ())),
                            preferred_element_type=jnp.float32),
            0.0,
        )

    return pl.pallas_call(
        body,
        out_shape=jax.ShapeDtypeStruct((N, D), jnp.float32),
        grid=(N // _BLK,),
        in_specs=[
            pl.BlockSpec((_BLK, D), lambda i: (i, 0)),
            pl.BlockSpec((D, D), lambda i: (0, 0)),
        ],
        out_specs=pl.BlockSpec((_BLK, D), lambda i: (i, 0)),
    )(h, wc)


def _rms(t, g, eps=1e-5):
    inv = lax.rsqrt(jnp.mean(t * t, axis=-1, keepdims=True) + eps)
    return t * inv * g


def _tc_update(h, a0, a1, gc, wh, gh):
    def body(h_ref, a0_ref, a1_ref, gc_ref, wh_ref, gh_ref, o_ref):
        t = _rms(h_ref[...] + a0_ref[...] + a1_ref[...], gc_ref[...])
        y = jnp.maximum(
            lax.dot_general(t, wh_ref[...], (((1,), (1,)), ((), ())),
                            preferred_element_type=jnp.float32),
            0.0,
        )
        o_ref[...] = _rms(t + y, gh_ref[...])

    return pl.pallas_call(
        body,
        out_shape=jax.ShapeDtypeStruct((N, D), jnp.float32),
        grid=(N // _BLK,),
        in_specs=[
            pl.BlockSpec((_BLK, D), lambda i: (i, 0)),
            pl.BlockSpec((_BLK, D), lambda i: (i, 0)),
            pl.BlockSpec((_BLK, D), lambda i: (i, 0)),
            pl.BlockSpec((1, D), lambda i: (0, 0)),
            pl.BlockSpec((D, D), lambda i: (0, 0)),
            pl.BlockSpec((1, D), lambda i: (0, 0)),
        ],
        out_specs=pl.BlockSpec((_BLK, D), lambda i: (i, 0)),
    )(h, a0, a1, gc, wh, gh)


def _tc_fused(h, a0, a1, gc, wh, gh, wc_next):
    def body(h_ref, a0_ref, a1_ref, gc_ref, wh_ref, gh_ref, wc_ref,
             oh_ref, om_ref):
        t = _rms(h_ref[...] + a0_ref[...] + a1_ref[...], gc_ref[...])
        y = jnp.maximum(
            lax.dot_general(t, wh_ref[...], (((1,), (1,)), ((), ())),
                            preferred_element_type=jnp.float32),
            0.0,
        )
        hn = _rms(t + y, gh_ref[...])
        oh_ref[...] = hn
        om_ref[...] = jnp.maximum(
            lax.dot_general(hn, wc_ref[...], (((1,), (1,)), ((), ())),
                            preferred_element_type=jnp.float32),
            0.0,
        )

    return pl.pallas_call(
        body,
        out_shape=[jax.ShapeDtypeStruct((N, D), jnp.float32),
                   jax.ShapeDtypeStruct((N, D), jnp.float32)],
        grid=(N // _BLK,),
        in_specs=[
            pl.BlockSpec((_BLK, D), lambda i: (i, 0)),
            pl.BlockSpec((_BLK, D), lambda i: (i, 0)),
            pl.BlockSpec((_BLK, D), lambda i: (i, 0)),
            pl.BlockSpec((1, D), lambda i: (0, 0)),
            pl.BlockSpec((D, D), lambda i: (0, 0)),
            pl.BlockSpec((1, D), lambda i: (0, 0)),
            pl.BlockSpec((D, D), lambda i: (0, 0)),
        ],
        out_specs=[pl.BlockSpec((_BLK, D), lambda i: (i, 0)),
                   pl.BlockSpec((_BLK, D), lambda i: (i, 0))],
    )(h, a0, a1, gc, wh, gh, wc_next)


def kernel(x, edge_index, Wc, gc, Wh, gh):
    zrows = jnp.zeros((ROWS_PER_TILE, D), jnp.float32)

    n_layers = Wc.shape[0]
    h = x
    m = _tc_msg(h, Wc[0])
    for l in range(n_layers):
        a0, a1 = _sc_segment_sum(m, edge_index, zrows)
        if l + 1 < n_layers:
            h, m = _tc_fused(h, a0, a1, gc[l].reshape(1, D), Wh[l],
                             gh[l].reshape(1, D), Wc[l + 1])
        else:
            h = _tc_update(h, a0, a1, gc[l].reshape(1, D), Wh[l],
                           gh[l].reshape(1, D))
    return h


# speedup vs baseline: 1.0024x; 1.0024x over previous
import functools

import jax
import jax.numpy as jnp
from jax import lax
from jax.experimental import pallas as pl
from jax.experimental.pallas import tpu as pltpu
from jax.experimental.pallas import tpu_sc as plsc

N = 10000
D = 128
E = 320000
NC = 2
NS = 16
CHUNK = 128
CPW = 80
ROWS_PER_TILE = 632
N_ACC = NS * ROWS_PER_TILE
LAST_TILE_OUT = N - (NS - 1) * ROWS_PER_TILE
NBUF = 2
IDEPTH = 4
N_REAL_CHUNKS = E // CHUNK

_MESH = plsc.VectorSubcoreMesh(core_axis_name="c", subcore_axis_name="s")


def _sc_segment_sum(m, ei, zrows):
    @functools.partial(
        pl.kernel,
        mesh=_MESH,
        out_type=[jax.ShapeDtypeStruct((N, D), jnp.float32),
                  jax.ShapeDtypeStruct((N, D), jnp.float32)],
        scratch_types=[
            pltpu.VMEM((IDEPTH, CHUNK), jnp.int32),
            pltpu.VMEM((IDEPTH, CHUNK), jnp.int32),
            pltpu.VMEM((NBUF, CHUNK, D), jnp.float32),
            pltpu.VMEM_SHARED((N_ACC, D), jnp.float32),
        ]
        + [pltpu.SemaphoreType.DMA] * (2 * IDEPTH + 2 * NBUF),
    )
    def k(m_hbm, ei_hbm, z_hbm, out0_hbm, out1_hbm, cbuf, rbuf,
          rows, acc, *sems):
        csem = sems[:IDEPTH]
        rsem = sems[IDEPTH:2 * IDEPTH]
        gsem = sems[2 * IDEPTH:2 * IDEPTH + NBUF]
        ssem = sems[2 * IDEPTH + NBUF:]
        c = lax.axis_index("c")
        s = lax.axis_index("s")
        wid = c * NS + s
        row_v = ei_hbm.at[0]
        col_v = ei_hbm.at[1]
        chunk0 = wid * CPW

        def real(q):
            return chunk0 + q < N_REAL_CHUNKS

        def colload(q, i):
            return pltpu.make_async_copy(
                col_v.at[pl.ds((chunk0 + q) * CHUNK, CHUNK)], cbuf.at[i],
                csem[i])

        def rowload(q, i):
            return pltpu.make_async_copy(
                row_v.at[pl.ds((chunk0 + q) * CHUNK, CHUNK)], rbuf.at[i],
                rsem[i])


        def gather(i, b):
            return pltpu.make_async_copy(m_hbm.at[cbuf.at[i]], rows.at[b],
                                         gsem[b])

        def scatter(i, b):
            return pltpu.make_async_copy(rows.at[b], acc.at[rbuf.at[i]],
                                         ssem[b])

        with jax.named_scope("sc_zero_and_prefetch"):
            z = pltpu.make_async_copy(
                z_hbm, acc.at[pl.ds(s * ROWS_PER_TILE, ROWS_PER_TILE)],
                ssem[1])
            z.start()

            for i in range(IDEPTH):
                colload(i, i).start()
                rowload(i, i).start()

            colload(0, 0).wait()
            gather(0, 0).start()

        def _loop_body(ci):
            for j in range(IDEPTH):
                q = ci + j
                b = j % NBUF
                bp = (j + 1) % NBUF
                i = j
                ip = (j + 1) % IDEPTH

                @pl.when((q >= 1) & real(q - 1))
                def _():
                    scatter((j - 1) % IDEPTH, bp).wait()

                    @pl.when((q + 3 < CPW) & real(q + 3))
                    def _():
                        rowload(q + 3, (j - 1) % IDEPTH).start()

                @pl.when((q + 1 < CPW) & real(q + 1))
                def _():
                    colload(q + 1, ip).wait()
                    gather(ip, bp).start()

                @pl.when(real(q))
                def _():
                    gather(i, b).wait()

                @pl.when((q + 4 < CPW) & real(q + 4))
                def _():
                    colload(q + 4, i).start()

                @pl.when(real(q))
                def _():
                    rowload(q, i).wait()

                @pl.when(q == 0)
                def _():
                    pltpu.make_async_copy(
                        z_hbm,
                        acc.at[pl.ds(s * ROWS_PER_TILE, ROWS_PER_TILE)],
                        ssem[1]).wait()
                    plsc.subcore_barrier()

                @pl.when(real(q))
                def _():
                    scatter(i, b).start(add=True)

        with jax.named_scope("sc_edge_loop"):

            @pl.loop(0, CPW, step=IDEPTH)
            def _(ci):
                _loop_body(ci)

            @pl.when(real(CPW - 1))
            def _():
                scatter((CPW - 1) % IDEPTH, (CPW - 1) % NBUF).wait()

        plsc.subcore_barrier()

        with jax.named_scope("sc_copy_out"):
            for cc, out_c in ((0, out0_hbm), (1, out1_hbm)):

                @pl.when((c == cc) & (s < NS - 1))
                def _():
                    pltpu.sync_copy(
                        acc.at[pl.ds(s * ROWS_PER_TILE, ROWS_PER_TILE)],
                        out_c.at[pl.ds(s * ROWS_PER_TILE, ROWS_PER_TILE)],
                    )

                @pl.when((c == cc) & (s == NS - 1))
                def _():
                    pltpu.sync_copy(
                        acc.at[pl.ds((NS - 1) * ROWS_PER_TILE, LAST_TILE_OUT)],
                        out_c.at[pl.ds((NS - 1) * ROWS_PER_TILE,
                                       LAST_TILE_OUT)],
                    )

    return k(m, ei, zrows)


_BLK = 2000


def _tc_msg(h, wc):
    def body(h_ref, w_ref, o_ref):
        o_ref[...] = jnp.maximum(
            lax.dot_general(h_ref[...], w_ref[...],
                            (((1,), (1,)), ((), ())),
                            preferred_element_type=jnp.float32),
            0.0,
        )

    return pl.pallas_call(
        body,
        out_shape=jax.ShapeDtypeStruct((N, D), jnp.float32),
        grid=(N // _BLK,),
        in_specs=[
            pl.BlockSpec((_BLK, D), lambda i: (i, 0)),
            pl.BlockSpec((D, D), lambda i: (0, 0)),
        ],
        out_specs=pl.BlockSpec((_BLK, D), lambda i: (i, 0)),
    )(h, wc)


def _rms(t, g, eps=1e-5):
    inv = lax.rsqrt(jnp.mean(t * t, axis=-1, keepdims=True) + eps)
    return t * inv * g


def _tc_update(h, a0, a1, gc, wh, gh):
    def body(h_ref, a0_ref, a1_ref, gc_ref, wh_ref, gh_ref, o_ref):
        t = _rms(h_ref[...] + a0_ref[...] + a1_ref[...], gc_ref[...])
        y = jnp.maximum(
            lax.dot_general(t, wh_ref[...], (((1,), (1,)), ((), ())),
                            preferred_element_type=jnp.float32),
            0.0,
        )
        o_ref[...] = _rms(t + y, gh_ref[...])

    return pl.pallas_call(
        body,
        out_shape=jax.ShapeDtypeStruct((N, D), jnp.float32),
        grid=(N // _BLK,),
        in_specs=[
            pl.BlockSpec((_BLK, D), lambda i: (i, 0)),
            pl.BlockSpec((_BLK, D), lambda i: (i, 0)),
            pl.BlockSpec((_BLK, D), lambda i: (i, 0)),
            pl.BlockSpec((1, D), lambda i: (0, 0)),
            pl.BlockSpec((D, D), lambda i: (0, 0)),
            pl.BlockSpec((1, D), lambda i: (0, 0)),
        ],
        out_specs=pl.BlockSpec((_BLK, D), lambda i: (i, 0)),
    )(h, a0, a1, gc, wh, gh)


def _tc_fused(h, a0, a1, gc, wh, gh, wc_next):
    def body(h_ref, a0_ref, a1_ref, gc_ref, wh_ref, gh_ref, wc_ref,
             oh_ref, om_ref):
        t = _rms(h_ref[...] + a0_ref[...] + a1_ref[...], gc_ref[...])
        y = jnp.maximum(
            lax.dot_general(t, wh_ref[...], (((1,), (1,)), ((), ())),
                            preferred_element_type=jnp.float32),
            0.0,
        )
        hn = _rms(t + y, gh_ref[...])
        oh_ref[...] = hn
        om_ref[...] = jnp.maximum(
            lax.dot_general(hn, wc_ref[...], (((1,), (1,)), ((), ())),
                            preferred_element_type=jnp.float32),
            0.0,
        )

    return pl.pallas_call(
        body,
        out_shape=[jax.ShapeDtypeStruct((N, D), jnp.float32),
                   jax.ShapeDtypeStruct((N, D), jnp.float32)],
        grid=(N // _BLK,),
        in_specs=[
            pl.BlockSpec((_BLK, D), lambda i: (i, 0)),
            pl.BlockSpec((_BLK, D), lambda i: (i, 0)),
            pl.BlockSpec((_BLK, D), lambda i: (i, 0)),
            pl.BlockSpec((1, D), lambda i: (0, 0)),
            pl.BlockSpec((D, D), lambda i: (0, 0)),
            pl.BlockSpec((1, D), lambda i: (0, 0)),
            pl.BlockSpec((D, D), lambda i: (0, 0)),
        ],
        out_specs=[pl.BlockSpec((_BLK, D), lambda i: (i, 0)),
                   pl.BlockSpec((_BLK, D), lambda i: (i, 0))],
    )(h, a0, a1, gc, wh, gh, wc_next)


def kernel(x, edge_index, Wc, gc, Wh, gh):
    zrows = jnp.zeros((ROWS_PER_TILE, D), jnp.float32)

    n_layers = Wc.shape[0]
    h = x
    m = _tc_msg(h, Wc[0])
    for l in range(n_layers):
        a0, a1 = _sc_segment_sum(m, edge_index, zrows)
        if l + 1 < n_layers:
            h, m = _tc_fused(h, a0, a1, gc[l].reshape(1, D), Wh[l],
                             gh[l].reshape(1, D), Wc[l + 1])
        else:
            h = _tc_update(h, a0, a1, gc[l].reshape(1, D), Wh[l],
                           gh[l].reshape(1, D))
    return h
